# Initial kernel scaffold; baseline (speedup 1.0000x reference)
import jax
import jax.numpy as jnp
from jax.experimental import pallas as pl


def kernel(input_batch, W):
    raise NotImplementedError("write your pallas kernel here")


# speedup vs baseline: 1.0426x; 1.0426x over previous
import functools

import jax
import jax.numpy as jnp
from jax.experimental import pallas as pl
from jax.experimental.pallas import tpu as pltpu

_NUM_EXPERTS = 64
_TOP_K = 8
_BT = 512


def _router_body(x_ref, w_ref, p_ref, i_ref):
    x = x_ref[...]
    w = w_ref[...]
    logits = jnp.dot(x, w, preferred_element_type=jnp.float32)
    m = jnp.max(logits, axis=1, keepdims=True)
    e = jnp.exp(logits - m)
    p = e / jnp.sum(e, axis=1, keepdims=True)

    idx = jax.lax.broadcasted_iota(jnp.int32, p.shape, 1)
    vals = p
    for j in range(_TOP_K):
        mj = jnp.max(vals, axis=1, keepdims=True)
        cand = jnp.where(vals == mj, idx, _NUM_EXPERTS)
        aj = jnp.min(cand, axis=1, keepdims=True)
        p_ref[:, j : j + 1] = mj
        i_ref[:, j : j + 1] = aj
        vals = jnp.where(idx == aj, -1.0, vals)


def kernel(input_batch, W):
    n_tokens, d_model = input_batch.shape
    grid = (n_tokens // _BT,)
    p_out, i_out = pl.pallas_call(
        _router_body,
        grid=grid,
        in_specs=[
            pl.BlockSpec((_BT, d_model), lambda i: (i, 0)),
            pl.BlockSpec((d_model, _NUM_EXPERTS), lambda i: (0, 0)),
        ],
        out_specs=[
            pl.BlockSpec((_BT, _TOP_K), lambda i: (i, 0)),
            pl.BlockSpec((_BT, _TOP_K), lambda i: (i, 0)),
        ],
        out_shape=[
            jax.ShapeDtypeStruct((n_tokens, _TOP_K), jnp.float32),
            jax.ShapeDtypeStruct((n_tokens, _TOP_K), jnp.int32),
        ],
        compiler_params=pltpu.CompilerParams(
            dimension_semantics=("arbitrary",),
        ),
    )(input_batch, W)
    return (p_out, i_out)


# speedup vs baseline: 1.2300x; 1.1797x over previous
import functools

import jax
import jax.numpy as jnp
from jax.experimental import pallas as pl
from jax.experimental.pallas import tpu as pltpu

_NUM_EXPERTS = 64
_TOP_K = 8
_BT = 1024


def _router_body(x_ref, w_ref, p_ref, i_ref):
    x = x_ref[...]
    w = w_ref[...]
    logits = jnp.dot(x, w, preferred_element_type=jnp.float32)
    m = jnp.max(logits, axis=1, keepdims=True)
    e = jnp.exp(logits - m)
    p = e / jnp.sum(e, axis=1, keepdims=True)

    idx = jax.lax.broadcasted_iota(jnp.int32, p.shape, 1)
    vals = p
    for j in range(_TOP_K):
        mj = jnp.max(vals, axis=1, keepdims=True)
        cand = jnp.where(vals == mj, idx, _NUM_EXPERTS)
        aj = jnp.min(cand, axis=1, keepdims=True)
        p_ref[:, j : j + 1] = mj
        i_ref[:, j : j + 1] = aj
        vals = jnp.where(idx == aj, -1.0, vals)


def kernel(input_batch, W):
    n_tokens, d_model = input_batch.shape
    grid = (n_tokens // _BT,)
    p_out, i_out = pl.pallas_call(
        _router_body,
        grid=grid,
        in_specs=[
            pl.BlockSpec((_BT, d_model), lambda i: (i, 0)),
            pl.BlockSpec((d_model, _NUM_EXPERTS), lambda i: (0, 0)),
        ],
        out_specs=[
            pl.BlockSpec((_BT, _TOP_K), lambda i: (i, 0)),
            pl.BlockSpec((_BT, _TOP_K), lambda i: (i, 0)),
        ],
        out_shape=[
            jax.ShapeDtypeStruct((n_tokens, _TOP_K), jnp.float32),
            jax.ShapeDtypeStruct((n_tokens, _TOP_K), jnp.int32),
        ],
        compiler_params=pltpu.CompilerParams(
            dimension_semantics=("arbitrary",),
        ),
    )(input_batch, W)
    return (p_out, i_out)


# speedup vs baseline: 1.2410x; 1.0090x over previous
import functools

import jax
import jax.numpy as jnp
from jax.experimental import pallas as pl
from jax.experimental.pallas import tpu as pltpu

_NUM_EXPERTS = 64
_TOP_K = 8
_BT = 2048


def _router_body(x_ref, w_ref, p_ref, i_ref):
    x = x_ref[...]
    w = w_ref[...]
    logits = jnp.dot(x, w, preferred_element_type=jnp.float32)
    m = jnp.max(logits, axis=1, keepdims=True)
    e = jnp.exp(logits - m)
    p = e / jnp.sum(e, axis=1, keepdims=True)

    idx = jax.lax.broadcasted_iota(jnp.int32, p.shape, 1)
    vals = p
    for j in range(_TOP_K):
        mj = jnp.max(vals, axis=1, keepdims=True)
        cand = jnp.where(vals == mj, idx, _NUM_EXPERTS)
        aj = jnp.min(cand, axis=1, keepdims=True)
        p_ref[:, j : j + 1] = mj
        i_ref[:, j : j + 1] = aj
        vals = jnp.where(idx == aj, -1.0, vals)


def kernel(input_batch, W):
    n_tokens, d_model = input_batch.shape
    grid = (n_tokens // _BT,)
    p_out, i_out = pl.pallas_call(
        _router_body,
        grid=grid,
        in_specs=[
            pl.BlockSpec((_BT, d_model), lambda i: (i, 0)),
            pl.BlockSpec((d_model, _NUM_EXPERTS), lambda i: (0, 0)),
        ],
        out_specs=[
            pl.BlockSpec((_BT, _TOP_K), lambda i: (i, 0)),
            pl.BlockSpec((_BT, _TOP_K), lambda i: (i, 0)),
        ],
        out_shape=[
            jax.ShapeDtypeStruct((n_tokens, _TOP_K), jnp.float32),
            jax.ShapeDtypeStruct((n_tokens, _TOP_K), jnp.int32),
        ],
        compiler_params=pltpu.CompilerParams(
            dimension_semantics=("arbitrary",),
        ),
    )(input_batch, W)
    return (p_out, i_out)


# speedup vs baseline: 1.5027x; 1.2108x over previous
import jax
import jax.numpy as jnp
from jax.experimental import pallas as pl
from jax.experimental.pallas import tpu as pltpu

_NUM_EXPERTS = 64
_TOP_K = 8
_BT = 2048
_IDX_MASK = _NUM_EXPERTS - 1


def _router_body(x_ref, w_ref, p_ref, i_ref):
    x = x_ref[...]
    w = w_ref[...]
    logits = jnp.dot(x, w, preferred_element_type=jnp.float32)
    m = jnp.max(logits, axis=1, keepdims=True)
    e = jnp.exp(logits - m)
    denom = jnp.sum(e, axis=1, keepdims=True)

    idx = jax.lax.broadcasted_iota(jnp.int32, e.shape, 1)
    eb = jax.lax.bitcast_convert_type(e, jnp.int32)
    key = (eb & jnp.int32(~_IDX_MASK)) | (jnp.int32(_IDX_MASK) - idx)

    cols = []
    for _ in range(_TOP_K):
        kj = jnp.max(key, axis=1, keepdims=True)
        cols.append(kj)
        key = jnp.where(key == kj, jnp.int32(-(2**31)), key)
    ks = jnp.concatenate(cols, axis=1)

    sel_e = jax.lax.bitcast_convert_type(ks & jnp.int32(~_IDX_MASK), jnp.float32)
    p_ref[...] = sel_e / denom
    i_ref[...] = jnp.int32(_IDX_MASK) - (ks & jnp.int32(_IDX_MASK))


def kernel(input_batch, W):
    n_tokens, d_model = input_batch.shape
    grid = (n_tokens // _BT,)
    p_out, i_out = pl.pallas_call(
        _router_body,
        grid=grid,
        in_specs=[
            pl.BlockSpec((_BT, d_model), lambda i: (i, 0)),
            pl.BlockSpec((d_model, _NUM_EXPERTS), lambda i: (0, 0)),
        ],
        out_specs=[
            pl.BlockSpec((_BT, _TOP_K), lambda i: (i, 0)),
            pl.BlockSpec((_BT, _TOP_K), lambda i: (i, 0)),
        ],
        out_shape=[
            jax.ShapeDtypeStruct((n_tokens, _TOP_K), jnp.float32),
            jax.ShapeDtypeStruct((n_tokens, _TOP_K), jnp.int32),
        ],
        compiler_params=pltpu.CompilerParams(
            dimension_semantics=("arbitrary",),
        ),
    )(input_batch, W)
    return (p_out, i_out)


# speedup vs baseline: 1.7529x; 1.1665x over previous
import jax
import jax.numpy as jnp
from jax.experimental import pallas as pl
from jax.experimental.pallas import tpu as pltpu

_NUM_EXPERTS = 64
_TOP_K = 8
_BT = 2048
_IDX_MASK = _NUM_EXPERTS - 1


def _router_body(x_ref, w_ref, p_ref, i_ref):
    x = x_ref[...]
    w = w_ref[...]
    logits = jnp.dot(x, w, preferred_element_type=jnp.float32)
    m = jnp.max(logits, axis=1, keepdims=True)
    e = jnp.exp(logits - m)
    denom = jnp.sum(e, axis=1, keepdims=True)

    idx = jax.lax.broadcasted_iota(jnp.int32, e.shape, 1)
    eb = jax.lax.bitcast_convert_type(e, jnp.int32)
    key = jax.lax.bitcast_convert_type(
        (eb & jnp.int32(~_IDX_MASK)) | (jnp.int32(_IDX_MASK) - idx), jnp.float32
    )

    cols = []
    for _ in range(_TOP_K):
        kj = jnp.max(key, axis=1, keepdims=True)
        cols.append(kj)
        key = jnp.where(key == kj, jnp.float32(-1.0), key)
    ks = jax.lax.bitcast_convert_type(
        jnp.concatenate(cols, axis=1), jnp.int32
    )

    sel_e = jax.lax.bitcast_convert_type(ks & jnp.int32(~_IDX_MASK), jnp.float32)
    p_ref[...] = sel_e / denom
    i_ref[...] = jnp.int32(_IDX_MASK) - (ks & jnp.int32(_IDX_MASK))


def kernel(input_batch, W):
    n_tokens, d_model = input_batch.shape
    grid = (n_tokens // _BT,)
    p_out, i_out = pl.pallas_call(
        _router_body,
        grid=grid,
        in_specs=[
            pl.BlockSpec((_BT, d_model), lambda i: (i, 0)),
            pl.BlockSpec((d_model, _NUM_EXPERTS), lambda i: (0, 0)),
        ],
        out_specs=[
            pl.BlockSpec((_BT, _TOP_K), lambda i: (i, 0)),
            pl.BlockSpec((_BT, _TOP_K), lambda i: (i, 0)),
        ],
        out_shape=[
            jax.ShapeDtypeStruct((n_tokens, _TOP_K), jnp.float32),
            jax.ShapeDtypeStruct((n_tokens, _TOP_K), jnp.int32),
        ],
        compiler_params=pltpu.CompilerParams(
            dimension_semantics=("arbitrary",),
        ),
    )(input_batch, W)
    return (p_out, i_out)
